# baseline (speedup 1.0000x reference)
import functools

import jax
import jax.numpy as jnp
from jax import lax
from jax.experimental import pallas as pl
from jax.experimental.pallas import tpu as pltpu
from jax.experimental.pallas import tpu_sc as plsc

NEG_SLOPE = 0.2
_BLK = 2000
_K = 80
_NBUF = 5
_LANES = 16


def _lrelu(e):
    return jnp.where(e >= 0, e, e * NEG_SLOPE)


def _first_body(x_ref, w_ref, wp_ref, asrc_ref, adst_ref, hs_ref, hb_ref,
                av_ref, bv_ref):
    h = jnp.dot(x_ref[...], w_ref[...], preferred_element_type=jnp.float32)
    hb = jnp.dot(x_ref[...], wp_ref[...], preferred_element_type=jnp.float32)
    hh = h.shape[1] // 2
    hs_ref[0] = h[:, :hh]
    hs_ref[1] = h[:, hh:]
    hb_ref[0] = hb[:, :hh].astype(jnp.bfloat16)
    hb_ref[1] = hb[:, hh:].astype(jnp.bfloat16)
    av_ref[...] = jnp.sum(h * asrc_ref[...][None, :], axis=1, keepdims=True)
    bv_ref[...] = jnp.sum(h * adst_ref[...][None, :], axis=1, keepdims=True)


def _tc_first(x, W, Wp, a_src, a_dst):
    n, d = x.shape
    h = W.shape[1]
    return pl.pallas_call(
        _first_body,
        grid=(n // _BLK,),
        in_specs=[
            pl.BlockSpec((_BLK, d), lambda i: (i, 0)),
            pl.BlockSpec((d, h), lambda i: (0, 0)),
            pl.BlockSpec((d, h), lambda i: (0, 0)),
            pl.BlockSpec((h,), lambda i: (0,)),
            pl.BlockSpec((h,), lambda i: (0,)),
        ],
        out_specs=[
            pl.BlockSpec((2, _BLK, h // 2), lambda i: (0, i, 0)),
            pl.BlockSpec((2, _BLK, h // 2), lambda i: (0, i, 0)),
            pl.BlockSpec((_BLK, 1), lambda i: (i, 0)),
            pl.BlockSpec((_BLK, 1), lambda i: (i, 0)),
        ],
        out_shape=[
            jax.ShapeDtypeStruct((2, n, h // 2), jnp.float32),
            jax.ShapeDtypeStruct((2, n, h // 2), jnp.bfloat16),
            jax.ShapeDtypeStruct((n, 1), jnp.float32),
            jax.ShapeDtypeStruct((n, 1), jnp.float32),
        ],
    )(x, W, Wp, a_src, a_dst)


def _combine(acc_ref, den_ref, hs_ref, as_ref, ad_ref, b_ref):
    h = jnp.concatenate([hs_ref[0], hs_ref[1]], axis=1)
    acc = jnp.concatenate([acc_ref[0], acc_ref[1]], axis=1)
    ws = jnp.exp(_lrelu(as_ref[...] + ad_ref[...]))
    num = acc + ws * h
    den = den_ref[...] + ws
    return num / den + b_ref[...][None, :]


def _mid_body(acc_ref, den_ref, hs_ref, as_ref, ad_ref, b_ref, w_ref, wp_ref,
              ansrc_ref, andst_ref, hn_ref, hb_ref, avn_ref, bvn_ref):
    out = jnp.maximum(_combine(acc_ref, den_ref, hs_ref, as_ref, ad_ref, b_ref), 0.0)
    hn = jnp.dot(out, w_ref[...], preferred_element_type=jnp.float32)
    hb = jnp.dot(out, wp_ref[...], preferred_element_type=jnp.float32)
    hh = hn.shape[1] // 2
    hn_ref[0] = hn[:, :hh]
    hn_ref[1] = hn[:, hh:]
    hb_ref[0] = hb[:, :hh].astype(jnp.bfloat16)
    hb_ref[1] = hb[:, hh:].astype(jnp.bfloat16)
    avn_ref[...] = jnp.sum(hn * ansrc_ref[...][None, :], axis=1, keepdims=True)
    bvn_ref[...] = jnp.sum(hn * andst_ref[...][None, :], axis=1, keepdims=True)


def _tc_mid(acc, den, hs_prev, asv, adv, b, W2, W2p, a_src2, a_dst2):
    _, n, hh = hs_prev.shape
    hdim = 2 * hh
    return pl.pallas_call(
        _mid_body,
        grid=(n // _BLK,),
        in_specs=[
            pl.BlockSpec((2, _BLK, hh), lambda i: (0, i, 0)),
            pl.BlockSpec((_BLK, 1), lambda i: (i, 0)),
            pl.BlockSpec((2, _BLK, hh), lambda i: (0, i, 0)),
            pl.BlockSpec((_BLK, 1), lambda i: (i, 0)),
            pl.BlockSpec((_BLK, 1), lambda i: (i, 0)),
            pl.BlockSpec((hdim,), lambda i: (0,)),
            pl.BlockSpec((hdim, hdim), lambda i: (0, 0)),
            pl.BlockSpec((hdim, hdim), lambda i: (0, 0)),
            pl.BlockSpec((hdim,), lambda i: (0,)),
            pl.BlockSpec((hdim,), lambda i: (0,)),
        ],
        out_specs=[
            pl.BlockSpec((2, _BLK, hh), lambda i: (0, i, 0)),
            pl.BlockSpec((2, _BLK, hh), lambda i: (0, i, 0)),
            pl.BlockSpec((_BLK, 1), lambda i: (i, 0)),
            pl.BlockSpec((_BLK, 1), lambda i: (i, 0)),
        ],
        out_shape=[
            jax.ShapeDtypeStruct((2, n, hh), jnp.float32),
            jax.ShapeDtypeStruct((2, n, hh), jnp.bfloat16),
            jax.ShapeDtypeStruct((n, 1), jnp.float32),
            jax.ShapeDtypeStruct((n, 1), jnp.float32),
        ],
    )(acc, den, hs_prev, asv, adv, b, W2, W2p, a_src2, a_dst2)


def _final_body(acc_ref, den_ref, hs_ref, as_ref, ad_ref, b_ref, lw_ref,
                lb_ref, out_ref):
    hid = _combine(acc_ref, den_ref, hs_ref, as_ref, ad_ref, b_ref)
    z = jnp.dot(hid, lw_ref[...], preferred_element_type=jnp.float32)
    z = z + lb_ref[...][None, :]
    m = jnp.max(z, axis=1, keepdims=True)
    lse = m + jnp.log(jnp.sum(jnp.exp(z - m), axis=1, keepdims=True))
    out_ref[...] = z - lse


def _tc_final(acc, den, hs_prev, asv, adv, b, linW, linb):
    _, n, hh = hs_prev.shape
    hdim = 2 * hh
    c = linW.shape[1]
    return pl.pallas_call(
        _final_body,
        grid=(n // _BLK,),
        in_specs=[
            pl.BlockSpec((2, _BLK, hh), lambda i: (0, i, 0)),
            pl.BlockSpec((_BLK, 1), lambda i: (i, 0)),
            pl.BlockSpec((2, _BLK, hh), lambda i: (0, i, 0)),
            pl.BlockSpec((_BLK, 1), lambda i: (i, 0)),
            pl.BlockSpec((_BLK, 1), lambda i: (i, 0)),
            pl.BlockSpec((hdim,), lambda i: (0,)),
            pl.BlockSpec((hdim, c), lambda i: (0, 0)),
            pl.BlockSpec((c,), lambda i: (0,)),
        ],
        out_specs=pl.BlockSpec((_BLK, c), lambda i: (i, 0)),
        out_shape=jax.ShapeDtypeStruct((n, c), jnp.float32),
    )(acc, den, hs_prev, asv, adv, b, linW, linb)


@functools.lru_cache(maxsize=None)
def _make_sc_edge(n, hdim, e):
    info = plsc.get_sparse_core_info()
    nc, ns = info.num_cores, info.num_subcores
    hh = hdim // nc
    ept = e // ns
    nch = ept // _K
    rpt = ((n // ns) // _K + 1) * _K
    rlast = n - (ns - 1) * rpt
    mesh = plsc.VectorSubcoreMesh(core_axis_name="c", subcore_axis_name="s")

    @functools.partial(
        pl.kernel,
        out_type=(jax.ShapeDtypeStruct((nc, n, hh), jnp.float32),
                  jax.ShapeDtypeStruct((1, n), jnp.float32)),
        mesh=mesh,
        compiler_params=pltpu.CompilerParams(needs_layout_passes=False,
                                             use_tc_tiling_on_sc=False),
        scratch_types=[
            pltpu.VMEM((nch, _K), jnp.int32),
            pltpu.VMEM((nch, _K), jnp.int32),
            pltpu.VMEM((n,), jnp.float32),
            pltpu.VMEM((n,), jnp.float32),
            [pltpu.VMEM((_K, hh), jnp.bfloat16)] * _NBUF,
            pltpu.VMEM((_K, hh), jnp.float32),
            [pltpu.VMEM((_K,), jnp.float32)] * _NBUF,
            pltpu.VMEM((n // 5,), jnp.float32),
            pltpu.VMEM_SHARED((n, hh), jnp.float32),
            pltpu.VMEM_SHARED((n,), jnp.float32),
            [pltpu.SemaphoreType.DMA] * _NBUF,
            [pltpu.SemaphoreType.DMA] * _NBUF,
        ],
    )
    def sc_edge(h_hbm, as_hbm, ad_hbm, src_hbm, dst_hbm,
                acc_hbm, den_hbm,
                src_v, dst_v, as_v, ad_v, gbufs, obuf, wbufs,
                zeros_v, acc_sh, den_sh, gsems, ssems):
        zero16 = jnp.full((_LANES,), 0.0, jnp.float32)
        cid = lax.axis_index("c")
        sid = lax.axis_index("s")

        pltpu.sync_copy(as_hbm, as_v)
        pltpu.sync_copy(ad_hbm, ad_v)
        pltpu.sync_copy(src_hbm.at[sid], src_v)
        pltpu.sync_copy(dst_hbm.at[sid], dst_v)

        def _zden(i, carry):
            zeros_v[pl.ds(pl.multiple_of(i * _LANES, _LANES), _LANES)] = zero16
            return carry
        lax.fori_loop(0, n // 5 // _LANES, _zden, 0)

        def _zrow(i, carry):
            r = i // (hh // _LANES)
            col = (i % (hh // _LANES)) * _LANES
            obuf[r, pl.ds(pl.multiple_of(col, _LANES), _LANES)] = zero16
            return carry
        lax.fori_loop(0, _K * hh // _LANES, _zrow, 0)

        base = sid * rpt

        def _zero_rows(cnt):
            for k in range(cnt // _K):
                pltpu.sync_copy(obuf, acc_sh.at[pl.ds(base + k * _K, _K)])
            rem = cnt % _K
            if rem:
                pltpu.sync_copy(obuf.at[pl.ds(0, rem)],
                                acc_sh.at[pl.ds(base + (cnt // _K) * _K, rem)])

        @pl.when(sid < ns - 1)
        def _():
            _zero_rows(rpt)

        @pl.when(sid == ns - 1)
        def _():
            _zero_rows(rlast)

        @pl.when(sid == 0)
        def _():
            for k in range(5):
                pltpu.sync_copy(zeros_v, den_sh.at[pl.ds(k * (n // 5), n // 5)])

        plsc.subcore_barrier()

        def _start_gather(ci, buf, sem):
            pltpu.async_copy(h_hbm.at[cid].at[src_v.at[ci]], buf, sem)

        def _wait_gather(ci, buf, sem):
            pltpu.make_async_copy(h_hbm.at[cid].at[src_v.at[ci]], buf, sem).wait()

        def _compute_w(ci, wbuf):
            ws = []
            for o in range(_K // _LANES):
                s16 = src_v[ci, pl.ds(o * _LANES, _LANES)]
                d16 = dst_v[ci, pl.ds(o * _LANES, _LANES)]
                ev = plsc.load_gather(as_v, [s16]) + plsc.load_gather(ad_v, [d16])
                w16 = jnp.exp(_lrelu(ev))
                wbuf[pl.ds(o * _LANES, _LANES)] = w16
                ws.append(w16)
            return ws

        hi_mask = jnp.full((_LANES,), -65536, jnp.int32)

        def _scale(gbuf, obuf, ws):
            for o in range(_K // _LANES):
                for j2 in range(_LANES):
                    wj = ws[o][j2]
                    j = o * _LANES + j2
                    for g in range(hh // (2 * _LANES)):
                        v = plsc.bitcast(gbuf[j, pl.ds(g * 2 * _LANES, 2 * _LANES)],
                                         jnp.int32)
                        lo = plsc.bitcast(v << 16, jnp.float32)
                        hi = plsc.bitcast(v & hi_mask, jnp.float32)
                        obuf[j, pl.ds(g * 2 * _LANES, _LANES)] = lo * wj
                        obuf[j, pl.ds(g * 2 * _LANES + _LANES, _LANES)] = hi * wj

        def _start_scatter(ci, buf, wbuf, ssem):
            pltpu.async_copy(buf, acc_sh.at[dst_v.at[ci]], ssem, add=True)

            @pl.when(cid == 0)
            def _():
                pltpu.async_copy(wbuf, den_sh.at[dst_v.at[ci]], ssem, add=True)

        def _wait_scatter(ci, buf, wbuf, ssem):
            pltpu.make_async_copy(buf, acc_sh.at[dst_v.at[ci]], ssem).wait()

            @pl.when(cid == 0)
            def _():
                pltpu.make_async_copy(wbuf, den_sh.at[dst_v.at[ci]], ssem).wait()

        for b in range(_NBUF - 1):
            _start_gather(b, gbufs[b], gsems[b])

        def _round(k, carry):
            c0 = _NBUF * k
            for b in range(_NBUF):
                ci = c0 + b
                bp = (b - 1) % _NBUF
                cip = ci - 1
                _start_gather(jnp.where(cip + _NBUF < nch, cip + _NBUF, 0),
                              gbufs[bp], gsems[bp])
                ws = _compute_w(ci, wbufs[b])
                _wait_gather(ci, gbufs[b], gsems[b])
                if b == 0:
                    @pl.when(k > 0)
                    def _():
                        _wait_scatter(cip, obuf, wbufs[bp], ssems[bp])
                else:
                    _wait_scatter(cip, obuf, wbufs[bp], ssems[bp])
                _scale(gbufs[b], obuf, ws)
                _start_scatter(ci, obuf, wbufs[b], ssems[b])
            return carry
        lax.fori_loop(0, nch // _NBUF, _round, 0)
        _wait_scatter(nch - 1, obuf, wbufs[_NBUF - 1], ssems[_NBUF - 1])
        for b in range(_NBUF - 1):
            _wait_gather(0, gbufs[b], gsems[b])

        plsc.subcore_barrier()

        @pl.when(sid < ns - 1)
        def _():
            pltpu.sync_copy(acc_sh.at[pl.ds(base, rpt)],
                            acc_hbm.at[cid, pl.ds(base, rpt)])

        @pl.when(sid == ns - 1)
        def _():
            pltpu.sync_copy(acc_sh.at[pl.ds(base, rlast)],
                            acc_hbm.at[cid, pl.ds(base, rlast)])

        @pl.when(jnp.logical_and(cid == 0, sid == 0))
        def _():
            pltpu.sync_copy(den_sh, den_hbm.at[0])

    return sc_edge


def kernel(x, edge_index, W1, a_src1, a_dst1, b1, W2, a_src2, a_dst2, b2,
           linW, linb):
    n, _ = x.shape
    e = edge_index.shape[1]
    hdim = W1.shape[1]
    info = plsc.get_sparse_core_info()
    ns = info.num_subcores
    src3d = edge_index[0].reshape(ns, e // (ns * _K), _K)
    dst3d = edge_index[1].reshape(ns, e // (ns * _K), _K)
    sc_edge = _make_sc_edge(n, hdim, e)

    perm = []
    for c in range(2):
        for g in range(hdim // 2 // (2 * _LANES)):
            bs = (hdim // 2) * c + 2 * _LANES * g
            for i in range(_LANES):
                perm += [bs + i, bs + _LANES + i]
    perm = jnp.array(perm, dtype=jnp.int32)
    W1p = W1[:, perm]
    W2p = W2[:, perm]

    hs1, hb1, as1, ad1 = _tc_first(x, W1, W1p, a_src1, a_dst1)
    acc1, den1 = sc_edge(hb1, as1.reshape(n), ad1.reshape(n), src3d, dst3d)
    hs2, hb2, as2, ad2 = _tc_mid(acc1, den1.T, hs1, as1, ad1, b1, W2, W2p,
                                 a_src2, a_dst2)
    acc2, den2 = sc_edge(hb2, as2.reshape(n), ad2.reshape(n), src3d, dst3d)
    logits = _tc_final(acc2, den2.T, hs2, as2, ad2, b2, linW, linb)
    return (logits, edge_index)


# speedup vs baseline: 1.2750x; 1.2750x over previous
import functools

import jax
import jax.numpy as jnp
from jax import lax
from jax.experimental import pallas as pl
from jax.experimental.pallas import tpu as pltpu
from jax.experimental.pallas import tpu_sc as plsc

NEG_SLOPE = 0.2
_BLK = 2000
_K = 32
_NBUF = 5
_LANES = 16


def _lrelu(e):
    return jnp.where(e >= 0, e, e * NEG_SLOPE)


def _first_body(x_ref, w_ref, wp_ref, asrc_ref, adst_ref, hs_ref, hb_ref,
                av_ref, bv_ref):
    h = jnp.dot(x_ref[...], w_ref[...], preferred_element_type=jnp.float32)
    hb = jnp.dot(x_ref[...], wp_ref[...], preferred_element_type=jnp.float32)
    hh = h.shape[1] // 2
    hs_ref[0] = h[:, :hh]
    hs_ref[1] = h[:, hh:]
    hb_ref[0] = hb[:, :hh].astype(jnp.bfloat16)
    hb_ref[1] = hb[:, hh:].astype(jnp.bfloat16)
    av_ref[...] = jnp.sum(h * asrc_ref[...][None, :], axis=1, keepdims=True)
    bv_ref[...] = jnp.sum(h * adst_ref[...][None, :], axis=1, keepdims=True)


def _tc_first(x, W, Wp, a_src, a_dst):
    n, d = x.shape
    h = W.shape[1]
    return pl.pallas_call(
        _first_body,
        grid=(n // _BLK,),
        in_specs=[
            pl.BlockSpec((_BLK, d), lambda i: (i, 0)),
            pl.BlockSpec((d, h), lambda i: (0, 0)),
            pl.BlockSpec((d, h), lambda i: (0, 0)),
            pl.BlockSpec((h,), lambda i: (0,)),
            pl.BlockSpec((h,), lambda i: (0,)),
        ],
        out_specs=[
            pl.BlockSpec((2, _BLK, h // 2), lambda i: (0, i, 0)),
            pl.BlockSpec((2, _BLK, h // 2), lambda i: (0, i, 0)),
            pl.BlockSpec((_BLK, 1), lambda i: (i, 0)),
            pl.BlockSpec((_BLK, 1), lambda i: (i, 0)),
        ],
        out_shape=[
            jax.ShapeDtypeStruct((2, n, h // 2), jnp.float32),
            jax.ShapeDtypeStruct((2, n, h // 2), jnp.bfloat16),
            jax.ShapeDtypeStruct((n, 1), jnp.float32),
            jax.ShapeDtypeStruct((n, 1), jnp.float32),
        ],
    )(x, W, Wp, a_src, a_dst)


def _combine(acc_ref, den_ref, hs_ref, as_ref, ad_ref, b_ref):
    h = jnp.concatenate([hs_ref[0], hs_ref[1]], axis=1)
    acc = jnp.concatenate([acc_ref[0], acc_ref[1]], axis=1)
    ws = jnp.exp(_lrelu(as_ref[...] + ad_ref[...]))
    num = acc + ws * h
    den = den_ref[...] + ws
    return num / den + b_ref[...][None, :]


def _mid_body(acc_ref, den_ref, hs_ref, as_ref, ad_ref, b_ref, w_ref, wp_ref,
              ansrc_ref, andst_ref, hn_ref, hb_ref, avn_ref, bvn_ref):
    out = jnp.maximum(_combine(acc_ref, den_ref, hs_ref, as_ref, ad_ref, b_ref), 0.0)
    hn = jnp.dot(out, w_ref[...], preferred_element_type=jnp.float32)
    hb = jnp.dot(out, wp_ref[...], preferred_element_type=jnp.float32)
    hh = hn.shape[1] // 2
    hn_ref[0] = hn[:, :hh]
    hn_ref[1] = hn[:, hh:]
    hb_ref[0] = hb[:, :hh].astype(jnp.bfloat16)
    hb_ref[1] = hb[:, hh:].astype(jnp.bfloat16)
    avn_ref[...] = jnp.sum(hn * ansrc_ref[...][None, :], axis=1, keepdims=True)
    bvn_ref[...] = jnp.sum(hn * andst_ref[...][None, :], axis=1, keepdims=True)


def _tc_mid(acc, den, hs_prev, asv, adv, b, W2, W2p, a_src2, a_dst2):
    _, n, hh = hs_prev.shape
    hdim = 2 * hh
    return pl.pallas_call(
        _mid_body,
        grid=(n // _BLK,),
        in_specs=[
            pl.BlockSpec((2, _BLK, hh), lambda i: (0, i, 0)),
            pl.BlockSpec((_BLK, 1), lambda i: (i, 0)),
            pl.BlockSpec((2, _BLK, hh), lambda i: (0, i, 0)),
            pl.BlockSpec((_BLK, 1), lambda i: (i, 0)),
            pl.BlockSpec((_BLK, 1), lambda i: (i, 0)),
            pl.BlockSpec((hdim,), lambda i: (0,)),
            pl.BlockSpec((hdim, hdim), lambda i: (0, 0)),
            pl.BlockSpec((hdim, hdim), lambda i: (0, 0)),
            pl.BlockSpec((hdim,), lambda i: (0,)),
            pl.BlockSpec((hdim,), lambda i: (0,)),
        ],
        out_specs=[
            pl.BlockSpec((2, _BLK, hh), lambda i: (0, i, 0)),
            pl.BlockSpec((2, _BLK, hh), lambda i: (0, i, 0)),
            pl.BlockSpec((_BLK, 1), lambda i: (i, 0)),
            pl.BlockSpec((_BLK, 1), lambda i: (i, 0)),
        ],
        out_shape=[
            jax.ShapeDtypeStruct((2, n, hh), jnp.float32),
            jax.ShapeDtypeStruct((2, n, hh), jnp.bfloat16),
            jax.ShapeDtypeStruct((n, 1), jnp.float32),
            jax.ShapeDtypeStruct((n, 1), jnp.float32),
        ],
    )(acc, den, hs_prev, asv, adv, b, W2, W2p, a_src2, a_dst2)


def _final_body(acc_ref, den_ref, hs_ref, as_ref, ad_ref, b_ref, lw_ref,
                lb_ref, out_ref):
    hid = _combine(acc_ref, den_ref, hs_ref, as_ref, ad_ref, b_ref)
    z = jnp.dot(hid, lw_ref[...], preferred_element_type=jnp.float32)
    z = z + lb_ref[...][None, :]
    m = jnp.max(z, axis=1, keepdims=True)
    lse = m + jnp.log(jnp.sum(jnp.exp(z - m), axis=1, keepdims=True))
    out_ref[...] = z - lse


def _tc_final(acc, den, hs_prev, asv, adv, b, linW, linb):
    _, n, hh = hs_prev.shape
    hdim = 2 * hh
    c = linW.shape[1]
    return pl.pallas_call(
        _final_body,
        grid=(n // _BLK,),
        in_specs=[
            pl.BlockSpec((2, _BLK, hh), lambda i: (0, i, 0)),
            pl.BlockSpec((_BLK, 1), lambda i: (i, 0)),
            pl.BlockSpec((2, _BLK, hh), lambda i: (0, i, 0)),
            pl.BlockSpec((_BLK, 1), lambda i: (i, 0)),
            pl.BlockSpec((_BLK, 1), lambda i: (i, 0)),
            pl.BlockSpec((hdim,), lambda i: (0,)),
            pl.BlockSpec((hdim, c), lambda i: (0, 0)),
            pl.BlockSpec((c,), lambda i: (0,)),
        ],
        out_specs=pl.BlockSpec((_BLK, c), lambda i: (i, 0)),
        out_shape=jax.ShapeDtypeStruct((n, c), jnp.float32),
    )(acc, den, hs_prev, asv, adv, b, linW, linb)


@functools.lru_cache(maxsize=None)
def _make_sc_edge(n, hdim, e):
    info = plsc.get_sparse_core_info()
    nc, ns = info.num_cores, info.num_subcores
    hh = hdim // nc
    ept = e // ns
    nch = ept // _K
    rpt = ((n // ns) // _K + 1) * _K
    rlast = n - (ns - 1) * rpt
    mesh = plsc.VectorSubcoreMesh(core_axis_name="c", subcore_axis_name="s")

    @functools.partial(
        pl.kernel,
        out_type=(jax.ShapeDtypeStruct((nc, n, hh), jnp.float32),
                  jax.ShapeDtypeStruct((1, n), jnp.float32)),
        mesh=mesh,
        compiler_params=pltpu.CompilerParams(needs_layout_passes=False,
                                             use_tc_tiling_on_sc=False),
        scratch_types=[
            pltpu.VMEM((nch, _K), jnp.int32),
            pltpu.VMEM((nch, _K), jnp.int32),
            pltpu.VMEM((n,), jnp.float32),
            pltpu.VMEM((n,), jnp.float32),
            [pltpu.VMEM((_K, hh), jnp.bfloat16)] * _NBUF,
            [pltpu.VMEM((_K, hh), jnp.float32)] * _NBUF,
            [pltpu.VMEM((_K,), jnp.float32)] * _NBUF,
            pltpu.VMEM((n // 5,), jnp.float32),
            pltpu.VMEM_SHARED((n, hh), jnp.float32),
            pltpu.VMEM_SHARED((n,), jnp.float32),
            [pltpu.SemaphoreType.DMA] * _NBUF,
            [pltpu.SemaphoreType.DMA] * _NBUF,
        ],
    )
    def sc_edge(h_hbm, as_hbm, ad_hbm, src_hbm, dst_hbm,
                acc_hbm, den_hbm,
                src_v, dst_v, as_v, ad_v, gbufs, obufs, wbufs,
                zeros_v, acc_sh, den_sh, gsems, ssems):
        zero16 = jnp.full((_LANES,), 0.0, jnp.float32)
        cid = lax.axis_index("c")
        sid = lax.axis_index("s")

        pltpu.sync_copy(as_hbm, as_v)
        pltpu.sync_copy(ad_hbm, ad_v)
        pltpu.sync_copy(src_hbm.at[sid], src_v)
        pltpu.sync_copy(dst_hbm.at[sid], dst_v)

        def _zden(i, carry):
            zeros_v[pl.ds(pl.multiple_of(i * _LANES, _LANES), _LANES)] = zero16
            return carry
        lax.fori_loop(0, n // 5 // _LANES, _zden, 0)

        def _zrow(i, carry):
            r = i // (hh // _LANES)
            col = (i % (hh // _LANES)) * _LANES
            obufs[0][r, pl.ds(pl.multiple_of(col, _LANES), _LANES)] = zero16
            return carry
        lax.fori_loop(0, _K * hh // _LANES, _zrow, 0)

        base = sid * rpt

        def _zero_rows(cnt):
            for k in range(cnt // _K):
                pltpu.sync_copy(obufs[0], acc_sh.at[pl.ds(base + k * _K, _K)])
            rem = cnt % _K
            if rem:
                pltpu.sync_copy(obufs[0].at[pl.ds(0, rem)],
                                acc_sh.at[pl.ds(base + (cnt // _K) * _K, rem)])

        @pl.when(sid < ns - 1)
        def _():
            _zero_rows(rpt)

        @pl.when(sid == ns - 1)
        def _():
            _zero_rows(rlast)

        @pl.when(sid == 0)
        def _():
            for k in range(5):
                pltpu.sync_copy(zeros_v, den_sh.at[pl.ds(k * (n // 5), n // 5)])

        plsc.subcore_barrier()

        def _start_gather(ci, buf, sem):
            pltpu.async_copy(h_hbm.at[cid].at[src_v.at[ci]], buf, sem)

        def _wait_gather(ci, buf, sem):
            pltpu.make_async_copy(h_hbm.at[cid].at[src_v.at[ci]], buf, sem).wait()

        def _compute_w(ci, wbuf):
            ws = []
            for o in range(_K // _LANES):
                s16 = src_v[ci, pl.ds(o * _LANES, _LANES)]
                d16 = dst_v[ci, pl.ds(o * _LANES, _LANES)]
                ev = plsc.load_gather(as_v, [s16]) + plsc.load_gather(ad_v, [d16])
                w16 = jnp.exp(_lrelu(ev))
                wbuf[pl.ds(o * _LANES, _LANES)] = w16
                ws.append(w16)
            return ws

        hi_mask = jnp.full((_LANES,), -65536, jnp.int32)

        def _scale(gbuf, obuf, ws):
            for o in range(_K // _LANES):
                for j2 in range(_LANES):
                    wj = ws[o][j2]
                    j = o * _LANES + j2
                    for g in range(hh // (2 * _LANES)):
                        v = plsc.bitcast(gbuf[j, pl.ds(g * 2 * _LANES, 2 * _LANES)],
                                         jnp.int32)
                        lo = plsc.bitcast(v << 16, jnp.float32)
                        hi = plsc.bitcast(v & hi_mask, jnp.float32)
                        obuf[j, pl.ds(g * 2 * _LANES, _LANES)] = lo * wj
                        obuf[j, pl.ds(g * 2 * _LANES + _LANES, _LANES)] = hi * wj

        def _start_scatter(ci, buf, wbuf, ssem):
            pltpu.async_copy(buf, acc_sh.at[dst_v.at[ci]], ssem, add=True)

            @pl.when(cid == 0)
            def _():
                pltpu.async_copy(wbuf, den_sh.at[dst_v.at[ci]], ssem, add=True)

        def _wait_scatter(ci, buf, wbuf, ssem):
            pltpu.make_async_copy(buf, acc_sh.at[dst_v.at[ci]], ssem).wait()

            @pl.when(cid == 0)
            def _():
                pltpu.make_async_copy(wbuf, den_sh.at[dst_v.at[ci]], ssem).wait()

        for b in range(_NBUF - 1):
            _start_gather(b, gbufs[b], gsems[b])

        def _round(k, carry):
            c0 = _NBUF * k
            for b in range(_NBUF):
                ci = c0 + b
                bp = (b - 1) % _NBUF
                cip = ci - 1
                _start_gather(jnp.where(cip + _NBUF < nch, cip + _NBUF, 0),
                              gbufs[bp], gsems[bp])
                ws = _compute_w(ci, wbufs[b])
                _wait_gather(ci, gbufs[b], gsems[b])
                _scale(gbufs[b], obufs[b], ws)
                if b == 0:
                    @pl.when(k > 0)
                    def _():
                        _wait_scatter(cip, obufs[bp], wbufs[bp], ssems[bp])
                else:
                    _wait_scatter(cip, obufs[bp], wbufs[bp], ssems[bp])
                _start_scatter(ci, obufs[b], wbufs[b], ssems[b])
            return carry
        lax.fori_loop(0, nch // _NBUF, _round, 0)
        _wait_scatter(nch - 1, obufs[_NBUF - 1], wbufs[_NBUF - 1],
                      ssems[_NBUF - 1])
        for b in range(_NBUF - 1):
            _wait_gather(0, gbufs[b], gsems[b])

        plsc.subcore_barrier()

        @pl.when(sid < ns - 1)
        def _():
            pltpu.sync_copy(acc_sh.at[pl.ds(base, rpt)],
                            acc_hbm.at[cid, pl.ds(base, rpt)])

        @pl.when(sid == ns - 1)
        def _():
            pltpu.sync_copy(acc_sh.at[pl.ds(base, rlast)],
                            acc_hbm.at[cid, pl.ds(base, rlast)])

        @pl.when(jnp.logical_and(cid == 0, sid == 0))
        def _():
            pltpu.sync_copy(den_sh, den_hbm.at[0])

    return sc_edge


def kernel(x, edge_index, W1, a_src1, a_dst1, b1, W2, a_src2, a_dst2, b2,
           linW, linb):
    n, _ = x.shape
    e = edge_index.shape[1]
    hdim = W1.shape[1]
    info = plsc.get_sparse_core_info()
    ns = info.num_subcores
    src3d = edge_index[0].reshape(ns, e // (ns * _K), _K)
    dst3d = edge_index[1].reshape(ns, e // (ns * _K), _K)
    sc_edge = _make_sc_edge(n, hdim, e)

    perm = []
    for c in range(2):
        for g in range(hdim // 2 // (2 * _LANES)):
            bs = (hdim // 2) * c + 2 * _LANES * g
            for i in range(_LANES):
                perm += [bs + i, bs + _LANES + i]
    perm = jnp.array(perm, dtype=jnp.int32)
    W1p = W1[:, perm]
    W2p = W2[:, perm]

    hs1, hb1, as1, ad1 = _tc_first(x, W1, W1p, a_src1, a_dst1)
    acc1, den1 = sc_edge(hb1, as1.reshape(n), ad1.reshape(n), src3d, dst3d)
    hs2, hb2, as2, ad2 = _tc_mid(acc1, den1.T, hs1, as1, ad1, b1, W2, W2p,
                                 a_src2, a_dst2)
    acc2, den2 = sc_edge(hb2, as2.reshape(n), ad2.reshape(n), src3d, dst3d)
    logits = _tc_final(acc2, den2.T, hs2, as2, ad2, b2, linW, linb)
    return (logits, edge_index)


# speedup vs baseline: 1.2751x; 1.0001x over previous
import functools

import jax
import jax.numpy as jnp
from jax import lax
from jax.experimental import pallas as pl
from jax.experimental.pallas import tpu as pltpu
from jax.experimental.pallas import tpu_sc as plsc

NEG_SLOPE = 0.2
_BLK = 2000
_K = 32
_NBUF = 5
_LANES = 16


def _lrelu(e):
    return jnp.where(e >= 0, e, e * NEG_SLOPE)


def _first_body(x_ref, w_ref, wp_ref, asrc_ref, adst_ref, hs_ref, hb_ref,
                av_ref, bv_ref):
    h = jnp.dot(x_ref[...], w_ref[...], preferred_element_type=jnp.float32)
    hb = jnp.dot(x_ref[...], wp_ref[...], preferred_element_type=jnp.float32)
    hh = h.shape[1] // 2
    hs_ref[0] = h[:, :hh]
    hs_ref[1] = h[:, hh:]
    hb_ref[0] = hb[:, :hh].astype(jnp.bfloat16)
    hb_ref[1] = hb[:, hh:].astype(jnp.bfloat16)
    av_ref[...] = jnp.sum(h * asrc_ref[...][None, :], axis=1, keepdims=True)
    bv_ref[...] = jnp.sum(h * adst_ref[...][None, :], axis=1, keepdims=True)


def _tc_first(x, W, Wp, a_src, a_dst):
    n, d = x.shape
    h = W.shape[1]
    return pl.pallas_call(
        _first_body,
        grid=(n // _BLK,),
        in_specs=[
            pl.BlockSpec((_BLK, d), lambda i: (i, 0)),
            pl.BlockSpec((d, h), lambda i: (0, 0)),
            pl.BlockSpec((d, h), lambda i: (0, 0)),
            pl.BlockSpec((h,), lambda i: (0,)),
            pl.BlockSpec((h,), lambda i: (0,)),
        ],
        out_specs=[
            pl.BlockSpec((2, _BLK, h // 2), lambda i: (0, i, 0)),
            pl.BlockSpec((2, _BLK, h // 2), lambda i: (0, i, 0)),
            pl.BlockSpec((_BLK, 1), lambda i: (i, 0)),
            pl.BlockSpec((_BLK, 1), lambda i: (i, 0)),
        ],
        out_shape=[
            jax.ShapeDtypeStruct((2, n, h // 2), jnp.float32),
            jax.ShapeDtypeStruct((2, n, h // 2), jnp.bfloat16),
            jax.ShapeDtypeStruct((n, 1), jnp.float32),
            jax.ShapeDtypeStruct((n, 1), jnp.float32),
        ],
    )(x, W, Wp, a_src, a_dst)


def _combine(acc_ref, den_ref, hs_ref, as_ref, ad_ref, b_ref):
    h = jnp.concatenate([hs_ref[0], hs_ref[1]], axis=1)
    acc = jnp.concatenate([acc_ref[0], acc_ref[1]], axis=1)
    ws = jnp.exp(_lrelu(as_ref[...] + ad_ref[...]))
    num = acc + ws * h
    den = den_ref[...] + ws
    return num / den + b_ref[...][None, :]


def _mid_body(acc_ref, den_ref, hs_ref, as_ref, ad_ref, b_ref, w_ref, wp_ref,
              ansrc_ref, andst_ref, hn_ref, hb_ref, avn_ref, bvn_ref):
    out = jnp.maximum(_combine(acc_ref, den_ref, hs_ref, as_ref, ad_ref, b_ref), 0.0)
    hn = jnp.dot(out, w_ref[...], preferred_element_type=jnp.float32)
    hb = jnp.dot(out, wp_ref[...], preferred_element_type=jnp.float32)
    hh = hn.shape[1] // 2
    hn_ref[0] = hn[:, :hh]
    hn_ref[1] = hn[:, hh:]
    hb_ref[0] = hb[:, :hh].astype(jnp.bfloat16)
    hb_ref[1] = hb[:, hh:].astype(jnp.bfloat16)
    avn_ref[...] = jnp.sum(hn * ansrc_ref[...][None, :], axis=1, keepdims=True)
    bvn_ref[...] = jnp.sum(hn * andst_ref[...][None, :], axis=1, keepdims=True)


def _tc_mid(acc, den, hs_prev, asv, adv, b, W2, W2p, a_src2, a_dst2):
    _, n, hh = hs_prev.shape
    hdim = 2 * hh
    return pl.pallas_call(
        _mid_body,
        grid=(n // _BLK,),
        in_specs=[
            pl.BlockSpec((2, _BLK, hh), lambda i: (0, i, 0)),
            pl.BlockSpec((_BLK, 1), lambda i: (i, 0)),
            pl.BlockSpec((2, _BLK, hh), lambda i: (0, i, 0)),
            pl.BlockSpec((_BLK, 1), lambda i: (i, 0)),
            pl.BlockSpec((_BLK, 1), lambda i: (i, 0)),
            pl.BlockSpec((hdim,), lambda i: (0,)),
            pl.BlockSpec((hdim, hdim), lambda i: (0, 0)),
            pl.BlockSpec((hdim, hdim), lambda i: (0, 0)),
            pl.BlockSpec((hdim,), lambda i: (0,)),
            pl.BlockSpec((hdim,), lambda i: (0,)),
        ],
        out_specs=[
            pl.BlockSpec((2, _BLK, hh), lambda i: (0, i, 0)),
            pl.BlockSpec((2, _BLK, hh), lambda i: (0, i, 0)),
            pl.BlockSpec((_BLK, 1), lambda i: (i, 0)),
            pl.BlockSpec((_BLK, 1), lambda i: (i, 0)),
        ],
        out_shape=[
            jax.ShapeDtypeStruct((2, n, hh), jnp.float32),
            jax.ShapeDtypeStruct((2, n, hh), jnp.bfloat16),
            jax.ShapeDtypeStruct((n, 1), jnp.float32),
            jax.ShapeDtypeStruct((n, 1), jnp.float32),
        ],
    )(acc, den, hs_prev, asv, adv, b, W2, W2p, a_src2, a_dst2)


def _final_body(acc_ref, den_ref, hs_ref, as_ref, ad_ref, b_ref, lw_ref,
                lb_ref, out_ref):
    hid = _combine(acc_ref, den_ref, hs_ref, as_ref, ad_ref, b_ref)
    z = jnp.dot(hid, lw_ref[...], preferred_element_type=jnp.float32)
    z = z + lb_ref[...][None, :]
    m = jnp.max(z, axis=1, keepdims=True)
    lse = m + jnp.log(jnp.sum(jnp.exp(z - m), axis=1, keepdims=True))
    out_ref[...] = z - lse


def _tc_final(acc, den, hs_prev, asv, adv, b, linW, linb):
    _, n, hh = hs_prev.shape
    hdim = 2 * hh
    c = linW.shape[1]
    return pl.pallas_call(
        _final_body,
        grid=(n // _BLK,),
        in_specs=[
            pl.BlockSpec((2, _BLK, hh), lambda i: (0, i, 0)),
            pl.BlockSpec((_BLK, 1), lambda i: (i, 0)),
            pl.BlockSpec((2, _BLK, hh), lambda i: (0, i, 0)),
            pl.BlockSpec((_BLK, 1), lambda i: (i, 0)),
            pl.BlockSpec((_BLK, 1), lambda i: (i, 0)),
            pl.BlockSpec((hdim,), lambda i: (0,)),
            pl.BlockSpec((hdim, c), lambda i: (0, 0)),
            pl.BlockSpec((c,), lambda i: (0,)),
        ],
        out_specs=pl.BlockSpec((_BLK, c), lambda i: (i, 0)),
        out_shape=jax.ShapeDtypeStruct((n, c), jnp.float32),
    )(acc, den, hs_prev, asv, adv, b, linW, linb)


@functools.lru_cache(maxsize=None)
def _make_sc_edge(n, hdim, e):
    info = plsc.get_sparse_core_info()
    nc, ns = info.num_cores, info.num_subcores
    hh = hdim // nc
    ept = e // ns
    nch = ept // _K
    rpt = ((n // ns) // _K + 1) * _K
    rlast = n - (ns - 1) * rpt
    mesh = plsc.VectorSubcoreMesh(core_axis_name="c", subcore_axis_name="s")

    @functools.partial(
        pl.kernel,
        out_type=(jax.ShapeDtypeStruct((nc, n, hh), jnp.float32),
                  jax.ShapeDtypeStruct((1, n), jnp.float32)),
        mesh=mesh,
        compiler_params=pltpu.CompilerParams(needs_layout_passes=False,
                                             use_tc_tiling_on_sc=False),
        scratch_types=[
            pltpu.VMEM((nch, _K), jnp.int32),
            pltpu.VMEM((nch, _K), jnp.int32),
            pltpu.VMEM((n,), jnp.float32),
            pltpu.VMEM((n,), jnp.float32),
            [pltpu.VMEM((_K, hh), jnp.bfloat16)] * _NBUF,
            [pltpu.VMEM((_K, hh), jnp.float32)] * _NBUF,
            [pltpu.VMEM((_K,), jnp.float32)] * _NBUF,
            pltpu.VMEM((n // 5,), jnp.float32),
            pltpu.VMEM_SHARED((n, hh), jnp.float32),
            pltpu.VMEM_SHARED((n,), jnp.float32),
            [pltpu.SemaphoreType.DMA] * _NBUF,
            [pltpu.SemaphoreType.DMA] * _NBUF,
        ],
    )
    def sc_edge(h_hbm, as_hbm, ad_hbm, src_hbm, dst_hbm,
                acc_hbm, den_hbm,
                src_v, dst_v, as_v, ad_v, gbufs, obufs, wbufs,
                zeros_v, acc_sh, den_sh, gsems, ssems):
        zero16 = jnp.full((_LANES,), 0.0, jnp.float32)
        cid = lax.axis_index("c")
        sid = lax.axis_index("s")

        pltpu.sync_copy(as_hbm, as_v)
        pltpu.sync_copy(ad_hbm, ad_v)
        pltpu.sync_copy(src_hbm.at[sid], src_v)
        pltpu.sync_copy(dst_hbm.at[sid], dst_v)

        def _zden(i, carry):
            zeros_v[pl.ds(pl.multiple_of(i * _LANES, _LANES), _LANES)] = zero16
            return carry
        lax.fori_loop(0, n // 5 // _LANES, _zden, 0)

        def _zrow(i, carry):
            r = i // (hh // _LANES)
            col = (i % (hh // _LANES)) * _LANES
            obufs[0][r, pl.ds(pl.multiple_of(col, _LANES), _LANES)] = zero16
            return carry
        lax.fori_loop(0, _K * hh // _LANES, _zrow, 0)

        base = sid * rpt

        def _zero_rows(cnt):
            for k in range(cnt // _K):
                pltpu.sync_copy(obufs[0], acc_sh.at[pl.ds(base + k * _K, _K)])
            rem = cnt % _K
            if rem:
                pltpu.sync_copy(obufs[0].at[pl.ds(0, rem)],
                                acc_sh.at[pl.ds(base + (cnt // _K) * _K, rem)])

        @pl.when(sid < ns - 1)
        def _():
            _zero_rows(rpt)

        @pl.when(sid == ns - 1)
        def _():
            _zero_rows(rlast)

        @pl.when(sid == 0)
        def _():
            for k in range(5):
                pltpu.sync_copy(zeros_v, den_sh.at[pl.ds(k * (n // 5), n // 5)])

        plsc.subcore_barrier()

        def _start_gather(ci, buf, sem):
            half = _K // 2
            pltpu.async_copy(h_hbm.at[cid].at[src_v.at[ci].at[pl.ds(0, half)]],
                             buf.at[pl.ds(0, half)], sem)
            pltpu.async_copy(h_hbm.at[cid].at[src_v.at[ci].at[pl.ds(half, half)]],
                             buf.at[pl.ds(half, half)], sem)

        def _wait_gather(ci, buf, sem):
            half = _K // 2
            pltpu.make_async_copy(h_hbm.at[cid].at[src_v.at[ci].at[pl.ds(0, half)]],
                                  buf.at[pl.ds(0, half)], sem).wait()
            pltpu.make_async_copy(h_hbm.at[cid].at[src_v.at[ci].at[pl.ds(half, half)]],
                                  buf.at[pl.ds(half, half)], sem).wait()

        def _compute_w(ci, wbuf):
            ws = []
            for o in range(_K // _LANES):
                s16 = src_v[ci, pl.ds(o * _LANES, _LANES)]
                d16 = dst_v[ci, pl.ds(o * _LANES, _LANES)]
                ev = plsc.load_gather(as_v, [s16]) + plsc.load_gather(ad_v, [d16])
                w16 = jnp.exp(_lrelu(ev))
                wbuf[pl.ds(o * _LANES, _LANES)] = w16
                ws.append(w16)
            return ws

        hi_mask = jnp.full((_LANES,), -65536, jnp.int32)

        def _scale(gbuf, obuf, ws):
            for o in range(_K // _LANES):
                for j2 in range(_LANES):
                    wj = ws[o][j2]
                    j = o * _LANES + j2
                    for g in range(hh // (2 * _LANES)):
                        v = plsc.bitcast(gbuf[j, pl.ds(g * 2 * _LANES, 2 * _LANES)],
                                         jnp.int32)
                        lo = plsc.bitcast(v << 16, jnp.float32)
                        hi = plsc.bitcast(v & hi_mask, jnp.float32)
                        obuf[j, pl.ds(g * 2 * _LANES, _LANES)] = lo * wj
                        obuf[j, pl.ds(g * 2 * _LANES + _LANES, _LANES)] = hi * wj

        def _start_scatter(ci, buf, wbuf, ssem):
            pltpu.async_copy(buf, acc_sh.at[dst_v.at[ci]], ssem, add=True)

            @pl.when(cid == 0)
            def _():
                pltpu.async_copy(wbuf, den_sh.at[dst_v.at[ci]], ssem, add=True)

        def _wait_scatter(ci, buf, wbuf, ssem):
            pltpu.make_async_copy(buf, acc_sh.at[dst_v.at[ci]], ssem).wait()

            @pl.when(cid == 0)
            def _():
                pltpu.make_async_copy(wbuf, den_sh.at[dst_v.at[ci]], ssem).wait()

        for b in range(_NBUF - 1):
            _start_gather(b, gbufs[b], gsems[b])

        def _round(k, carry):
            c0 = _NBUF * k
            for b in range(_NBUF):
                ci = c0 + b
                bp = (b - 1) % _NBUF
                cip = ci - 1
                _start_gather(jnp.where(cip + _NBUF < nch, cip + _NBUF, 0),
                              gbufs[bp], gsems[bp])
                ws = _compute_w(ci, wbufs[b])
                _wait_gather(ci, gbufs[b], gsems[b])
                _scale(gbufs[b], obufs[b], ws)
                if b == 0:
                    @pl.when(k > 0)
                    def _():
                        _wait_scatter(cip, obufs[bp], wbufs[bp], ssems[bp])
                else:
                    _wait_scatter(cip, obufs[bp], wbufs[bp], ssems[bp])
                _start_scatter(ci, obufs[b], wbufs[b], ssems[b])
            return carry
        lax.fori_loop(0, nch // _NBUF, _round, 0)
        _wait_scatter(nch - 1, obufs[_NBUF - 1], wbufs[_NBUF - 1],
                      ssems[_NBUF - 1])
        for b in range(_NBUF - 1):
            _wait_gather(0, gbufs[b], gsems[b])

        plsc.subcore_barrier()

        @pl.when(sid < ns - 1)
        def _():
            pltpu.sync_copy(acc_sh.at[pl.ds(base, rpt)],
                            acc_hbm.at[cid, pl.ds(base, rpt)])

        @pl.when(sid == ns - 1)
        def _():
            pltpu.sync_copy(acc_sh.at[pl.ds(base, rlast)],
                            acc_hbm.at[cid, pl.ds(base, rlast)])

        @pl.when(jnp.logical_and(cid == 0, sid == 0))
        def _():
            pltpu.sync_copy(den_sh, den_hbm.at[0])

    return sc_edge


def kernel(x, edge_index, W1, a_src1, a_dst1, b1, W2, a_src2, a_dst2, b2,
           linW, linb):
    n, _ = x.shape
    e = edge_index.shape[1]
    hdim = W1.shape[1]
    info = plsc.get_sparse_core_info()
    ns = info.num_subcores
    src3d = edge_index[0].reshape(ns, e // (ns * _K), _K)
    dst3d = edge_index[1].reshape(ns, e // (ns * _K), _K)
    sc_edge = _make_sc_edge(n, hdim, e)

    perm = []
    for c in range(2):
        for g in range(hdim // 2 // (2 * _LANES)):
            bs = (hdim // 2) * c + 2 * _LANES * g
            for i in range(_LANES):
                perm += [bs + i, bs + _LANES + i]
    perm = jnp.array(perm, dtype=jnp.int32)
    W1p = W1[:, perm]
    W2p = W2[:, perm]

    hs1, hb1, as1, ad1 = _tc_first(x, W1, W1p, a_src1, a_dst1)
    acc1, den1 = sc_edge(hb1, as1.reshape(n), ad1.reshape(n), src3d, dst3d)
    hs2, hb2, as2, ad2 = _tc_mid(acc1, den1.T, hs1, as1, ad1, b1, W2, W2p,
                                 a_src2, a_dst2)
    acc2, den2 = sc_edge(hb2, as2.reshape(n), ad2.reshape(n), src3d, dst3d)
    logits = _tc_final(acc2, den2.T, hs2, as2, ad2, b2, linW, linb)
    return (logits, edge_index)
